# Initial kernel scaffold; baseline (speedup 1.0000x reference)
import jax
import jax.numpy as jnp
from jax.experimental import pallas as pl


def kernel(node_types, node_pos, adj_rand, node_rcg, Wih, Whh, bih, bhh, Wg, bg, Wm, Wdf1, bdf1, Wdf2, bdf2, Wfc1, bfc1, Wfc2, bfc2):
    raise NotImplementedError("write your pallas kernel here")


# speedup vs baseline: 2.6222x; 2.6222x over previous
import jax
import jax.numpy as jnp
from jax.experimental import pallas as pl
from jax.experimental.pallas import tpu as pltpu

_B = 256
_MAXN = 24
_NUM_TYPES = 10
_MAXPOS = 9
_HID = 301
_LAT = 56


def _kern(a_ref, x_ref, p_ref, pos_ref, rcg_ref,
          wih_r_ref, wih_z_ref, wih_n_ref,
          whh_r_ref, whh_z_ref, whh_n_ref,
          bih_r_ref, bih_z_ref, bih_n_ref,
          bhh_r_ref, bhh_z_ref, bhh_n_ref,
          wg_h_ref, wg_p_ref, bg_ref,
          wm_h_ref, wm_p_ref,
          wdf1_ref, bdf1_ref, wdf2_ref, bdf2_ref,
          wfc_h_ref, wfc_f_ref, bfc_ref,
          out_ref, g_ref):
    f32 = jnp.float32
    g_ref[...] = jnp.zeros((_MAXN, _B, _HID), f32)

    wih_r = wih_r_ref[...]
    wih_z = wih_z_ref[...]
    wih_n = wih_n_ref[...]
    whh_r = whh_r_ref[...]
    whh_z = whh_z_ref[...]
    whh_n = whh_n_ref[...]
    bih_r = bih_r_ref[...]
    bih_z = bih_z_ref[...]
    bih_n = bih_n_ref[...]
    bhh_r = bhh_r_ref[...]
    bhh_z = bhh_z_ref[...]
    bhh_n = bhh_n_ref[...]
    wg_h = wg_h_ref[...]
    wg_p = wg_p_ref[...]
    bg = bg_ref[...]
    wm_h = wm_h_ref[...]
    wm_p = wm_p_ref[...]

    u_iota = jax.lax.broadcasted_iota(jnp.int32, (_MAXN, _B), 0)

    def step(v, _):
        col = a_ref[v]
        m = jnp.where((col < 0.3) & (u_iota < v), 1.0, 0.0)
        hin = jnp.sum(m[:, :, None] * g_ref[...], axis=0)
        xv = x_ref[v]
        r = jax.nn.sigmoid(xv @ wih_r + bih_r + hin @ whh_r + bhh_r)
        z = jax.nn.sigmoid(xv @ wih_z + bih_z + hin @ whh_z + bhh_z)
        n = jnp.tanh(xv @ wih_n + bih_n + r * (hin @ whh_n + bhh_n))
        hv = (1.0 - z) * n + z * hin
        pv = p_ref[v]
        gate = jax.nn.sigmoid(hv @ wg_h + pv @ wg_p + bg)
        g_ref[v] = gate * (hv @ wm_h + pv @ wm_p)
        return hv

    hg = jax.lax.fori_loop(0, _MAXN, step, jnp.zeros((_B, _HID), f32))

    posq = pos_ref[...]
    j3 = jax.lax.broadcasted_iota(jnp.int32, (_B, _MAXN, 3 * _MAXPOS), 2)
    pj = j3 // 3
    kj = j3 - pj * 3
    niota = jax.lax.broadcasted_iota(jnp.int32, (_B, _MAXN, 3 * _MAXPOS), 1) + 1
    m27i = jnp.where(posq[:, :, None] == pj, niota, 0)
    nmax = jnp.max(m27i, axis=1)
    last = jnp.where((m27i == nmax[:, None, :]) & (m27i > 0), 1.0, 0.0)
    r3 = rcg_ref[...]
    rcg27 = (jnp.where(kj == 0, r3[:, :, 0:1], 0.0)
             + jnp.where(kj == 1, r3[:, :, 1:2], 0.0)
             + jnp.where(kj == 2, r3[:, :, 2:3], 0.0))
    df = jnp.sum(last * rcg27, axis=1)

    hdf = jnp.maximum(df @ wdf1_ref[...] + bdf1_ref[...], 0.0)
    hdf = hdf @ wdf2_ref[...] + bdf2_ref[...]

    out_ref[...] = hg @ wfc_h_ref[...] + (0.01 * hdf) @ wfc_f_ref[...] + bfc_ref[...]


def kernel(node_types, node_pos, adj_rand, node_rcg, Wih, Whh, bih, bhh,
           Wg, bg, Wm, Wdf1, bdf1, Wdf2, bdf2, Wfc1, bfc1, Wfc2, bfc2):
    f32 = jnp.float32
    H = _HID
    xt = jax.nn.one_hot(node_types, _NUM_TYPES, dtype=f32)
    xp = jax.nn.one_hot(node_pos, _MAXPOS, dtype=f32)
    x = jnp.concatenate([xt, xp], axis=-1).transpose(1, 0, 2)
    p = xp.transpose(1, 0, 2)
    a = adj_rand.transpose(2, 1, 0)

    args = (
        a, x, p, node_pos.astype(jnp.int32), node_rcg,
        Wih[0:H].T, Wih[H:2 * H].T, Wih[2 * H:].T,
        Whh[0:H].T, Whh[H:2 * H].T, Whh[2 * H:].T,
        bih[0:H][None, :], bih[H:2 * H][None, :], bih[2 * H:][None, :],
        bhh[0:H][None, :], bhh[H:2 * H][None, :], bhh[2 * H:][None, :],
        Wg[:, :H].T, Wg[:, H:].T, bg[None, :],
        Wm[:, :H].T, Wm[:, H:].T,
        Wdf1.T, bdf1[None, :], Wdf2.T, bdf2[None, :],
        jnp.concatenate([Wfc1[:, :H], Wfc2[:, :H]], axis=0).T,
        jnp.concatenate([Wfc1[:, H:], Wfc2[:, H:]], axis=0).T,
        jnp.concatenate([bfc1, bfc2])[None, :],
    )
    return pl.pallas_call(
        _kern,
        out_shape=jax.ShapeDtypeStruct((_B, 2 * _LAT), f32),
        scratch_shapes=[pltpu.VMEM((_MAXN, _B, _HID), f32)],
    )(*args)


# speedup vs baseline: 3.1111x; 1.1865x over previous
import jax
import jax.numpy as jnp
from jax.experimental import pallas as pl

_B = 256
_MAXN = 24
_NUM_TYPES = 10
_MAXPOS = 9
_HID = 301
_LAT = 56


def _kern(a_ref, x_ref, p_ref, pos_ref, rcg_ref,
          wih_r_ref, wih_z_ref, wih_n_ref,
          whh_r_ref, whh_z_ref, whh_n_ref,
          bih_r_ref, bih_z_ref, bih_n_ref,
          bhh_r_ref, bhh_z_ref, bhh_n_ref,
          wg_h_ref, wg_p_ref, bg_ref,
          wm_h_ref, wm_p_ref,
          wdf1_ref, bdf1_ref, wdf2_ref, bdf2_ref,
          wfc_h_ref, wfc_f_ref, bfc_ref,
          out_ref):
    f32 = jnp.float32
    wih_r = wih_r_ref[...]
    wih_z = wih_z_ref[...]
    wih_n = wih_n_ref[...]
    whh_r = whh_r_ref[...]
    whh_z = whh_z_ref[...]
    whh_n = whh_n_ref[...]
    bih_r = bih_r_ref[...]
    bih_z = bih_z_ref[...]
    bih_n = bih_n_ref[...]
    bhh_r = bhh_r_ref[...]
    bhh_z = bhh_z_ref[...]
    bhh_n = bhh_n_ref[...]
    wg_h = wg_h_ref[...]
    wg_p = wg_p_ref[...]
    bg = bg_ref[...]
    wm_h = wm_h_ref[...]
    wm_p = wm_p_ref[...]

    grows = []
    hv = None
    for v in range(_MAXN):
        if v == 0:
            hin = jnp.zeros((_B, _HID), f32)
        else:
            col = a_ref[v]
            terms = [jnp.where(col[:, u:u + 1] < 0.3, grows[u], 0.0)
                     for u in range(v)]
            while len(terms) > 1:
                terms = [terms[i] + terms[i + 1] if i + 1 < len(terms)
                         else terms[i] for i in range(0, len(terms), 2)]
            hin = terms[0]
        xv = x_ref[v]
        r = jax.nn.sigmoid(xv @ wih_r + bih_r + hin @ whh_r + bhh_r)
        z = jax.nn.sigmoid(xv @ wih_z + bih_z + hin @ whh_z + bhh_z)
        n = jnp.tanh(xv @ wih_n + bih_n + r * (hin @ whh_n + bhh_n))
        hv = (1.0 - z) * n + z * hin
        if v < _MAXN - 1:
            pv = p_ref[v]
            gate = jax.nn.sigmoid(hv @ wg_h + pv @ wg_p + bg)
            grows.append(gate * (hv @ wm_h + pv @ wm_p))
    hg = hv

    posq = pos_ref[...]
    j3 = jax.lax.broadcasted_iota(jnp.int32, (_B, _MAXN, 3 * _MAXPOS), 2)
    pj = j3 // 3
    kj = j3 - pj * 3
    niota = jax.lax.broadcasted_iota(jnp.int32, (_B, _MAXN, 3 * _MAXPOS), 1) + 1
    m27i = jnp.where(posq[:, :, None] == pj, niota, 0)
    nmax = jnp.max(m27i, axis=1)
    last = jnp.where((m27i == nmax[:, None, :]) & (m27i > 0), 1.0, 0.0)
    r3 = rcg_ref[...]
    rcg27 = (jnp.where(kj == 0, r3[:, :, 0:1], 0.0)
             + jnp.where(kj == 1, r3[:, :, 1:2], 0.0)
             + jnp.where(kj == 2, r3[:, :, 2:3], 0.0))
    df = jnp.sum(last * rcg27, axis=1)

    hdf = jnp.maximum(df @ wdf1_ref[...] + bdf1_ref[...], 0.0)
    hdf = hdf @ wdf2_ref[...] + bdf2_ref[...]

    out_ref[...] = hg @ wfc_h_ref[...] + (0.01 * hdf) @ wfc_f_ref[...] + bfc_ref[...]


def kernel(node_types, node_pos, adj_rand, node_rcg, Wih, Whh, bih, bhh,
           Wg, bg, Wm, Wdf1, bdf1, Wdf2, bdf2, Wfc1, bfc1, Wfc2, bfc2):
    f32 = jnp.float32
    H = _HID
    xt = jax.nn.one_hot(node_types, _NUM_TYPES, dtype=f32)
    xp = jax.nn.one_hot(node_pos, _MAXPOS, dtype=f32)
    x = jnp.concatenate([xt, xp], axis=-1).transpose(1, 0, 2)
    p = xp.transpose(1, 0, 2)
    a = adj_rand.transpose(2, 0, 1)

    args = (
        a, x, p, node_pos.astype(jnp.int32), node_rcg,
        Wih[0:H].T, Wih[H:2 * H].T, Wih[2 * H:].T,
        Whh[0:H].T, Whh[H:2 * H].T, Whh[2 * H:].T,
        bih[0:H][None, :], bih[H:2 * H][None, :], bih[2 * H:][None, :],
        bhh[0:H][None, :], bhh[H:2 * H][None, :], bhh[2 * H:][None, :],
        Wg[:, :H].T, Wg[:, H:].T, bg[None, :],
        Wm[:, :H].T, Wm[:, H:].T,
        Wdf1.T, bdf1[None, :], Wdf2.T, bdf2[None, :],
        jnp.concatenate([Wfc1[:, :H], Wfc2[:, :H]], axis=0).T,
        jnp.concatenate([Wfc1[:, H:], Wfc2[:, H:]], axis=0).T,
        jnp.concatenate([bfc1, bfc2])[None, :],
    )
    return pl.pallas_call(
        _kern,
        out_shape=jax.ShapeDtypeStruct((_B, 2 * _LAT), f32),
    )(*args)
